# baseline (speedup 1.0000x reference)
import jax
import jax.numpy as jnp

N = 10000
K = 16


def _knn_idx_hi(verts, k):
    sq = jnp.sum(verts * verts, axis=-1)
    chunks = []
    step = 2000
    for s in range(0, verts.shape[0], step):
        q = verts[s:s + step]
        qb = q.astype(jnp.bfloat16).astype(jnp.float32)
        vb = verts.astype(jnp.bfloat16).astype(jnp.float32)
        d = (jnp.sum(q * q, axis=-1)[:, None] + sq[None, :]
             - 2.0 * jax.lax.dot(qb, vb.T, precision=jax.lax.Precision.HIGHEST))
        _, idx = jax.lax.top_k(-d, k)
        chunks.append(idx)
    return jnp.concatenate(chunks, axis=0)


def _mlp(x, W1, b1, g, bt, W2, b2):
    h = x @ W1 + b1
    mu = jnp.mean(h, axis=-1, keepdims=True)
    var = jnp.var(h, axis=-1, keepdims=True)
    h = (h - mu) / jnp.sqrt(var + 1e-5) * g + bt
    h = jax.nn.gelu(h)
    return h @ W2 + b2


def kernel(vertices, features, We1, be1, ge, bte, We2, be2, Wo1, bo1, go, bto, Wo2, bo2):
    v = vertices[0]
    f = features[0]
    idx = _knn_idx_hi(v, K)
    nb_feat = f[idx]
    self_feat = jnp.broadcast_to(f[:, None, :], (v.shape[0], K, f.shape[-1]))
    rel = v[idx] - v[:, None, :]
    e = jnp.concatenate([nb_feat, self_feat, rel], axis=-1)
    e = _mlp(e, We1, be1, ge, bte, We2, be2)
    agg = jnp.mean(e, axis=1)
    out = _mlp(agg, Wo1, bo1, go, bto, Wo2, bo2)
    return out[None]


# speedup vs baseline: 4.6472x; 4.6472x over previous
import functools

import jax
import jax.numpy as jnp
from jax import lax
from jax.experimental import pallas as pl
from jax.experimental.pallas import tpu as pltpu
from jax.experimental.pallas import tpu_sc as plsc

N = 10000
K = 16
IN_CH = 256
OUT_CH = 16
HID = 32
NHAT = 10240
QB = 256
NBLK = NHAT // QB
NW = 32
E = K * NHAT
EPW = E // NW
CH = 128
NCH = EPW // CH

_F32 = jnp.float32
_BF16 = jnp.bfloat16


def _proj_body(f_ref, v_ref, wnb_ref, wself_ref, wrel_ref, b1_ref,
               p_ref, s_ref):
    fb = f_ref[...].astype(_BF16)
    vb = v_ref[...].astype(_BF16)
    wnb = wnb_ref[...].astype(_BF16)
    wself = wself_ref[...].astype(_BF16)
    wrel = wrel_ref[...].astype(_BF16)
    vc = jnp.dot(vb, wrel, preferred_element_type=_F32)
    p_ref[...] = jnp.dot(fb, wnb, preferred_element_type=_F32) + vc
    s_ref[...] = (jnp.dot(fb, wself, preferred_element_type=_F32) - vc
                  + b1_ref[...])


def _proj(f_pad, v_pad, wnb, wself, wrel, b1):
    rb = 1024
    grid = NHAT // rb
    return pl.pallas_call(
        _proj_body,
        grid=(grid,),
        in_specs=[
            pl.BlockSpec((rb, IN_CH), lambda i: (i, 0)),
            pl.BlockSpec((rb, 3), lambda i: (i, 0)),
            pl.BlockSpec((IN_CH, HID), lambda i: (0, 0)),
            pl.BlockSpec((IN_CH, HID), lambda i: (0, 0)),
            pl.BlockSpec((3, HID), lambda i: (0, 0)),
            pl.BlockSpec((1, HID), lambda i: (0, 0)),
        ],
        out_specs=[
            pl.BlockSpec((rb, HID), lambda i: (i, 0)),
            pl.BlockSpec((rb, HID), lambda i: (i, 0)),
        ],
        out_shape=[
            jax.ShapeDtypeStruct((NHAT, HID), _F32),
            jax.ShapeDtypeStruct((NHAT, HID), _F32),
        ],
    )(f_pad, v_pad, wnb, wself, wrel, b1)


def _knn_body(q_ref, vt_ref, idx_ref):
    qx = q_ref[:, 0:1]
    qy = q_ref[:, 1:2]
    qz = q_ref[:, 2:3]
    vx = vt_ref[0:1, :]
    vy = vt_ref[1:2, :]
    vz = vt_ref[2:3, :]
    qsq = qx * qx + qy * qy + qz * qz
    vsq = vx * vx + vy * vy + vz * vz

    def rb(a):
        return a.astype(_BF16).astype(_F32)

    dot = rb(qx) * rb(vx) + rb(qy) * rb(vy) + rb(qz) * rb(vz)
    d = (qsq + vsq) - 2.0 * dot

    iota = lax.broadcasted_iota(jnp.int32, (QB, NHAT), 1)
    cols = []
    for t in range(K):
        m = jnp.min(d, axis=1, keepdims=True)
        cand = jnp.where(d <= m, iota, NHAT)
        j = jnp.min(cand, axis=1)
        cols.append(j)
        if t < K - 1:
            d = jnp.where(iota == j[:, None], jnp.inf, d)
    idx_ref[...] = jnp.stack(cols, axis=1)


def _knn(v_pad, vt):
    return pl.pallas_call(
        _knn_body,
        grid=(NBLK,),
        in_specs=[
            pl.BlockSpec((QB, 3), lambda i: (i, 0)),
            pl.BlockSpec((3, NHAT), lambda i: (0, 0)),
        ],
        out_specs=pl.BlockSpec((QB, K), lambda i: (i, 0)),
        out_shape=jax.ShapeDtypeStruct((NHAT, K), jnp.int32),
    )(v_pad, vt)


def _gather_rows(table, idxt):
    mesh = plsc.VectorSubcoreMesh(core_axis_name="c", subcore_axis_name="s")

    @functools.partial(
        pl.kernel,
        mesh=mesh,
        out_type=jax.ShapeDtypeStruct((E, HID), _F32),
        scratch_types=[
            pltpu.VMEM((EPW,), jnp.int32),
            pltpu.VMEM((CH, HID), _F32),
            pltpu.SemaphoreType.DMA,
        ],
        compiler_params=pltpu.CompilerParams(use_tc_tiling_on_sc=False),
    )
    def gk(idx_hbm, tab_hbm, out_hbm, idx_v, buf, sem):
        wid = lax.axis_index("s") * 2 + lax.axis_index("c")
        base = wid * EPW
        pltpu.sync_copy(idx_hbm.at[pl.ds(base, EPW)], idx_v)

        def body(j, carry):
            pltpu.async_copy(
                tab_hbm.at[idx_v.at[pl.ds(j * CH, CH)]], buf, sem).wait()
            pltpu.sync_copy(buf, out_hbm.at[pl.ds(base + j * CH, CH)])
            return carry

        lax.fori_loop(0, NCH, body, 0)

    return gk(idxt, table)


def _tail_body(g_ref, s_ref, we2_ref, be2_ref, ge_ref, bte_ref,
               wo1_ref, bo1_ref, go_ref, bto_ref, wo2_ref, bo2_ref,
               out_ref):
    s = s_ref[...]
    we2 = we2_ref[...].astype(_BF16)
    ge = ge_ref[...]
    bte = bte_ref[...]
    acc = jnp.zeros((QB, OUT_CH), _F32)
    for k in range(K):
        h = g_ref[k] + s
        mu = jnp.mean(h, axis=-1, keepdims=True)
        var = jnp.mean((h - mu) ** 2, axis=-1, keepdims=True)
        h = (h - mu) * lax.rsqrt(var + 1e-5) * ge + bte
        h = jax.nn.gelu(h)
        acc = acc + jnp.dot(h.astype(_BF16), we2, preferred_element_type=_F32)
    agg = acc * (1.0 / K) + be2_ref[...]

    t = jnp.dot(agg.astype(_BF16), wo1_ref[...].astype(_BF16),
                preferred_element_type=_F32) + bo1_ref[...]
    mu = jnp.mean(t, axis=-1, keepdims=True)
    var = jnp.mean((t - mu) ** 2, axis=-1, keepdims=True)
    t = (t - mu) * lax.rsqrt(var + 1e-5) * go_ref[...] + bto_ref[...]
    t = jax.nn.gelu(t)
    out_ref[...] = jnp.dot(t.astype(_BF16), wo2_ref[...].astype(_BF16),
                           preferred_element_type=_F32) + bo2_ref[...]


def _tail(g3, s_all, we2, be2, ge, bte, wo1, bo1, go, bto, wo2, bo2):
    full = lambda a, b: pl.BlockSpec((a, b), lambda i: (0, 0))
    return pl.pallas_call(
        _tail_body,
        grid=(NBLK,),
        in_specs=[
            pl.BlockSpec((K, QB, HID), lambda i: (0, i, 0)),
            pl.BlockSpec((QB, HID), lambda i: (i, 0)),
            full(HID, OUT_CH), full(1, OUT_CH), full(1, HID), full(1, HID),
            full(OUT_CH, HID), full(1, HID), full(1, HID), full(1, HID),
            full(HID, OUT_CH), full(1, OUT_CH),
        ],
        out_specs=pl.BlockSpec((QB, OUT_CH), lambda i: (i, 0)),
        out_shape=jax.ShapeDtypeStruct((NHAT, OUT_CH), _F32),
    )(g3, s_all, we2, be2, ge, bte, wo1, bo1, go, bto, wo2, bo2)


def kernel(vertices, features, We1, be1, ge, bte, We2, be2,
           Wo1, bo1, go, bto, Wo2, bo2):
    v = vertices[0]
    f = features[0]
    pad = NHAT - N
    v_pad = jnp.concatenate(
        [v, jnp.full((pad, 3), 100.0, _F32)], axis=0)
    f_pad = jnp.concatenate([f, jnp.zeros((pad, IN_CH), _F32)], axis=0)
    vt = v_pad.T

    wnb = We1[:IN_CH]
    wself = We1[IN_CH:2 * IN_CH]
    wrel = We1[2 * IN_CH:]

    p_tab, s_all = _proj(f_pad, v_pad, wnb, wself, wrel,
                         be1.reshape(1, HID))
    idx = _knn(v_pad, vt)
    idxt = idx.T.reshape(-1)
    g = _gather_rows(p_tab, idxt)
    g3 = g.reshape(K, NHAT, HID)
    out = _tail(g3, s_all, We2, be2.reshape(1, OUT_CH),
                ge.reshape(1, HID), bte.reshape(1, HID),
                Wo1, bo1.reshape(1, HID), go.reshape(1, HID),
                bto.reshape(1, HID), Wo2, bo2.reshape(1, OUT_CH))
    return out[:N][None]


# speedup vs baseline: 4.7237x; 1.0165x over previous
import functools

import jax
import jax.numpy as jnp
from jax import lax
from jax.experimental import pallas as pl
from jax.experimental.pallas import tpu as pltpu
from jax.experimental.pallas import tpu_sc as plsc

N = 10000
K = 16
IN_CH = 256
OUT_CH = 16
HID = 32
NHAT = 10240
QB = 256
NBLK = NHAT // QB
NW = 32
E = K * NHAT
EPW = E // NW
CH = 128
NCH = EPW // CH

_F32 = jnp.float32
_BF16 = jnp.bfloat16


def _proj_body(f_ref, v_ref, wnb_ref, wself_ref, wrel_ref, b1_ref,
               p_ref, s_ref):
    fb = f_ref[...].astype(_BF16)
    vb = v_ref[...].astype(_BF16)
    wnb = wnb_ref[...].astype(_BF16)
    wself = wself_ref[...].astype(_BF16)
    wrel = wrel_ref[...].astype(_BF16)
    vc = jnp.dot(vb, wrel, preferred_element_type=_F32)
    p_ref[...] = jnp.dot(fb, wnb, preferred_element_type=_F32) + vc
    s_ref[...] = (jnp.dot(fb, wself, preferred_element_type=_F32) - vc
                  + b1_ref[...])


def _proj(f_pad, v_pad, wnb, wself, wrel, b1):
    rb = 1024
    grid = NHAT // rb
    return pl.pallas_call(
        _proj_body,
        grid=(grid,),
        in_specs=[
            pl.BlockSpec((rb, IN_CH), lambda i: (i, 0)),
            pl.BlockSpec((rb, 3), lambda i: (i, 0)),
            pl.BlockSpec((IN_CH, HID), lambda i: (0, 0)),
            pl.BlockSpec((IN_CH, HID), lambda i: (0, 0)),
            pl.BlockSpec((3, HID), lambda i: (0, 0)),
            pl.BlockSpec((1, HID), lambda i: (0, 0)),
        ],
        out_specs=[
            pl.BlockSpec((rb, HID), lambda i: (i, 0)),
            pl.BlockSpec((rb, HID), lambda i: (i, 0)),
        ],
        out_shape=[
            jax.ShapeDtypeStruct((NHAT, HID), _F32),
            jax.ShapeDtypeStruct((NHAT, HID), _F32),
        ],
    )(f_pad, v_pad, wnb, wself, wrel, b1)


def _knn_body(q_ref, vt_ref, idx_ref):
    qx = q_ref[:, 0:1]
    qy = q_ref[:, 1:2]
    qz = q_ref[:, 2:3]
    vx = vt_ref[0:1, :]
    vy = vt_ref[1:2, :]
    vz = vt_ref[2:3, :]
    qsq = qx * qx + qy * qy + qz * qz
    vsq = vx * vx + vy * vy + vz * vz

    def rb(a):
        return a.astype(_BF16).astype(_F32)

    dot = rb(qx) * rb(vx) + rb(qy) * rb(vy) + rb(qz) * rb(vz)
    d = (qsq + vsq) - 2.0 * dot

    iota = lax.broadcasted_iota(jnp.int32, (QB, NHAT), 1)
    cols = []
    for t in range(K):
        j = jnp.argmin(d, axis=1).astype(jnp.int32)
        cols.append(j)
        if t < K - 1:
            d = jnp.where(iota == j[:, None], jnp.inf, d)
    idx_ref[...] = jnp.stack(cols, axis=1)


def _knn(v_pad, vt):
    return pl.pallas_call(
        _knn_body,
        grid=(NBLK,),
        in_specs=[
            pl.BlockSpec((QB, 3), lambda i: (i, 0)),
            pl.BlockSpec((3, NHAT), lambda i: (0, 0)),
        ],
        out_specs=pl.BlockSpec((QB, K), lambda i: (i, 0)),
        out_shape=jax.ShapeDtypeStruct((NHAT, K), jnp.int32),
    )(v_pad, vt)


def _gather_rows(table, idxt):
    mesh = plsc.VectorSubcoreMesh(core_axis_name="c", subcore_axis_name="s")

    @functools.partial(
        pl.kernel,
        mesh=mesh,
        out_type=jax.ShapeDtypeStruct((E, HID), _F32),
        scratch_types=[
            pltpu.VMEM((EPW,), jnp.int32),
            pltpu.VMEM((CH, HID), _F32),
            pltpu.SemaphoreType.DMA,
        ],
        compiler_params=pltpu.CompilerParams(use_tc_tiling_on_sc=False),
    )
    def gk(idx_hbm, tab_hbm, out_hbm, idx_v, buf, sem):
        wid = lax.axis_index("s") * 2 + lax.axis_index("c")
        base = wid * EPW
        pltpu.sync_copy(idx_hbm.at[pl.ds(base, EPW)], idx_v)

        def body(j, carry):
            pltpu.async_copy(
                tab_hbm.at[idx_v.at[pl.ds(j * CH, CH)]], buf, sem).wait()
            pltpu.sync_copy(buf, out_hbm.at[pl.ds(base + j * CH, CH)])
            return carry

        lax.fori_loop(0, NCH, body, 0)

    return gk(idxt, table)


def _tail_body(g_ref, s_ref, we2_ref, be2_ref, ge_ref, bte_ref,
               wo1_ref, bo1_ref, go_ref, bto_ref, wo2_ref, bo2_ref,
               out_ref):
    s = s_ref[...]
    we2 = we2_ref[...].astype(_BF16)
    ge = ge_ref[...]
    bte = bte_ref[...]
    acc = jnp.zeros((QB, OUT_CH), _F32)
    for k in range(K):
        h = g_ref[k] + s
        mu = jnp.mean(h, axis=-1, keepdims=True)
        var = jnp.mean((h - mu) ** 2, axis=-1, keepdims=True)
        h = (h - mu) * lax.rsqrt(var + 1e-5) * ge + bte
        h = jax.nn.gelu(h)
        acc = acc + jnp.dot(h.astype(_BF16), we2, preferred_element_type=_F32)
    agg = acc * (1.0 / K) + be2_ref[...]

    t = jnp.dot(agg.astype(_BF16), wo1_ref[...].astype(_BF16),
                preferred_element_type=_F32) + bo1_ref[...]
    mu = jnp.mean(t, axis=-1, keepdims=True)
    var = jnp.mean((t - mu) ** 2, axis=-1, keepdims=True)
    t = (t - mu) * lax.rsqrt(var + 1e-5) * go_ref[...] + bto_ref[...]
    t = jax.nn.gelu(t)
    out_ref[...] = jnp.dot(t.astype(_BF16), wo2_ref[...].astype(_BF16),
                           preferred_element_type=_F32) + bo2_ref[...]


def _tail(g3, s_all, we2, be2, ge, bte, wo1, bo1, go, bto, wo2, bo2):
    full = lambda a, b: pl.BlockSpec((a, b), lambda i: (0, 0))
    return pl.pallas_call(
        _tail_body,
        grid=(NBLK,),
        in_specs=[
            pl.BlockSpec((K, QB, HID), lambda i: (0, i, 0)),
            pl.BlockSpec((QB, HID), lambda i: (i, 0)),
            full(HID, OUT_CH), full(1, OUT_CH), full(1, HID), full(1, HID),
            full(OUT_CH, HID), full(1, HID), full(1, HID), full(1, HID),
            full(HID, OUT_CH), full(1, OUT_CH),
        ],
        out_specs=pl.BlockSpec((QB, OUT_CH), lambda i: (i, 0)),
        out_shape=jax.ShapeDtypeStruct((NHAT, OUT_CH), _F32),
    )(g3, s_all, we2, be2, ge, bte, wo1, bo1, go, bto, wo2, bo2)


def kernel(vertices, features, We1, be1, ge, bte, We2, be2,
           Wo1, bo1, go, bto, Wo2, bo2):
    v = vertices[0]
    f = features[0]
    pad = NHAT - N
    v_pad = jnp.concatenate(
        [v, jnp.full((pad, 3), 100.0, _F32)], axis=0)
    f_pad = jnp.concatenate([f, jnp.zeros((pad, IN_CH), _F32)], axis=0)
    vt = v_pad.T

    wnb = We1[:IN_CH]
    wself = We1[IN_CH:2 * IN_CH]
    wrel = We1[2 * IN_CH:]

    p_tab, s_all = _proj(f_pad, v_pad, wnb, wself, wrel,
                         be1.reshape(1, HID))
    idx = _knn(v_pad, vt)
    idxt = idx.T.reshape(-1)
    g = _gather_rows(p_tab, idxt)
    g3 = g.reshape(K, NHAT, HID)
    out = _tail(g3, s_all, We2, be2.reshape(1, OUT_CH),
                ge.reshape(1, HID), bte.reshape(1, HID),
                Wo1, bo1.reshape(1, HID), go.reshape(1, HID),
                bto.reshape(1, HID), Wo2, bo2.reshape(1, OUT_CH))
    return out[:N][None]


# speedup vs baseline: 4.7238x; 1.0000x over previous
import functools

import jax
import jax.numpy as jnp
from jax import lax
from jax.experimental import pallas as pl
from jax.experimental.pallas import tpu as pltpu
from jax.experimental.pallas import tpu_sc as plsc

N = 10000
K = 16
IN_CH = 256
OUT_CH = 16
HID = 32
NHAT = 10240
QB = 256
NBLK = NHAT // QB
NW = 32
E = K * NHAT
EPW = E // NW
CH = 128
NCH = EPW // CH

_F32 = jnp.float32
_BF16 = jnp.bfloat16


def _proj_body(f_ref, v_ref, wnb_ref, wself_ref, wrel_ref, b1_ref,
               p_ref, s_ref):
    fb = f_ref[...].astype(_BF16)
    vb = v_ref[...].astype(_BF16)
    wnb = wnb_ref[...].astype(_BF16)
    wself = wself_ref[...].astype(_BF16)
    wrel = wrel_ref[...].astype(_BF16)
    vc = jnp.dot(vb, wrel, preferred_element_type=_F32)
    p_ref[...] = jnp.dot(fb, wnb, preferred_element_type=_F32) + vc
    s_ref[...] = (jnp.dot(fb, wself, preferred_element_type=_F32) - vc
                  + b1_ref[...])


def _proj(f_pad, v_pad, wnb, wself, wrel, b1):
    rb = 1024
    grid = NHAT // rb
    return pl.pallas_call(
        _proj_body,
        grid=(grid,),
        in_specs=[
            pl.BlockSpec((rb, IN_CH), lambda i: (i, 0)),
            pl.BlockSpec((rb, 3), lambda i: (i, 0)),
            pl.BlockSpec((IN_CH, HID), lambda i: (0, 0)),
            pl.BlockSpec((IN_CH, HID), lambda i: (0, 0)),
            pl.BlockSpec((3, HID), lambda i: (0, 0)),
            pl.BlockSpec((1, HID), lambda i: (0, 0)),
        ],
        out_specs=[
            pl.BlockSpec((rb, HID), lambda i: (i, 0)),
            pl.BlockSpec((rb, HID), lambda i: (i, 0)),
        ],
        out_shape=[
            jax.ShapeDtypeStruct((NHAT, HID), _F32),
            jax.ShapeDtypeStruct((NHAT, HID), _F32),
        ],
        compiler_params=pltpu.CompilerParams(
            dimension_semantics=("parallel",)),
    )(f_pad, v_pad, wnb, wself, wrel, b1)


def _knn_body(q_ref, vt_ref, idx_ref):
    qx = q_ref[:, 0:1]
    qy = q_ref[:, 1:2]
    qz = q_ref[:, 2:3]
    vx = vt_ref[0:1, :]
    vy = vt_ref[1:2, :]
    vz = vt_ref[2:3, :]
    qsq = qx * qx + qy * qy + qz * qz
    vsq = vx * vx + vy * vy + vz * vz

    def rb(a):
        return a.astype(_BF16).astype(_F32)

    dot = rb(qx) * rb(vx) + rb(qy) * rb(vy) + rb(qz) * rb(vz)
    d = (qsq + vsq) - 2.0 * dot

    iota = lax.broadcasted_iota(jnp.int32, (QB, NHAT), 1)
    cols = []
    for t in range(K):
        j = jnp.argmin(d, axis=1).astype(jnp.int32)
        cols.append(j)
        if t < K - 1:
            d = jnp.where(iota == j[:, None], jnp.inf, d)
    idx_ref[...] = jnp.stack(cols, axis=1)


def _knn(v_pad, vt):
    return pl.pallas_call(
        _knn_body,
        grid=(NBLK,),
        in_specs=[
            pl.BlockSpec((QB, 3), lambda i: (i, 0)),
            pl.BlockSpec((3, NHAT), lambda i: (0, 0)),
        ],
        out_specs=pl.BlockSpec((QB, K), lambda i: (i, 0)),
        out_shape=jax.ShapeDtypeStruct((NHAT, K), jnp.int32),
        compiler_params=pltpu.CompilerParams(
            dimension_semantics=("parallel",)),
    )(v_pad, vt)


def _gather_rows(table, idxt):
    mesh = plsc.VectorSubcoreMesh(core_axis_name="c", subcore_axis_name="s")

    @functools.partial(
        pl.kernel,
        mesh=mesh,
        out_type=jax.ShapeDtypeStruct((E, HID), _F32),
        scratch_types=[
            pltpu.VMEM((EPW,), jnp.int32),
            pltpu.VMEM((CH, HID), _F32),
            pltpu.SemaphoreType.DMA,
        ],
        compiler_params=pltpu.CompilerParams(use_tc_tiling_on_sc=False),
    )
    def gk(idx_hbm, tab_hbm, out_hbm, idx_v, buf, sem):
        wid = lax.axis_index("s") * 2 + lax.axis_index("c")
        base = wid * EPW
        pltpu.sync_copy(idx_hbm.at[pl.ds(base, EPW)], idx_v)

        def body(j, carry):
            pltpu.async_copy(
                tab_hbm.at[idx_v.at[pl.ds(j * CH, CH)]], buf, sem).wait()
            pltpu.sync_copy(buf, out_hbm.at[pl.ds(base + j * CH, CH)])
            return carry

        lax.fori_loop(0, NCH, body, 0)

    return gk(idxt, table)


def _tail_body(g_ref, s_ref, we2_ref, be2_ref, ge_ref, bte_ref,
               wo1_ref, bo1_ref, go_ref, bto_ref, wo2_ref, bo2_ref,
               out_ref):
    s = s_ref[...]
    we2 = we2_ref[...].astype(_BF16)
    ge = ge_ref[...]
    bte = bte_ref[...]
    acc = jnp.zeros((QB, OUT_CH), _F32)
    for k in range(K):
        h = g_ref[k] + s
        mu = jnp.mean(h, axis=-1, keepdims=True)
        var = jnp.mean((h - mu) ** 2, axis=-1, keepdims=True)
        h = (h - mu) * lax.rsqrt(var + 1e-5) * ge + bte
        h = jax.nn.gelu(h)
        acc = acc + jnp.dot(h.astype(_BF16), we2, preferred_element_type=_F32)
    agg = acc * (1.0 / K) + be2_ref[...]

    t = jnp.dot(agg.astype(_BF16), wo1_ref[...].astype(_BF16),
                preferred_element_type=_F32) + bo1_ref[...]
    mu = jnp.mean(t, axis=-1, keepdims=True)
    var = jnp.mean((t - mu) ** 2, axis=-1, keepdims=True)
    t = (t - mu) * lax.rsqrt(var + 1e-5) * go_ref[...] + bto_ref[...]
    t = jax.nn.gelu(t)
    out_ref[...] = jnp.dot(t.astype(_BF16), wo2_ref[...].astype(_BF16),
                           preferred_element_type=_F32) + bo2_ref[...]


def _tail(g3, s_all, we2, be2, ge, bte, wo1, bo1, go, bto, wo2, bo2):
    full = lambda a, b: pl.BlockSpec((a, b), lambda i: (0, 0))
    return pl.pallas_call(
        _tail_body,
        grid=(NBLK,),
        in_specs=[
            pl.BlockSpec((K, QB, HID), lambda i: (0, i, 0)),
            pl.BlockSpec((QB, HID), lambda i: (i, 0)),
            full(HID, OUT_CH), full(1, OUT_CH), full(1, HID), full(1, HID),
            full(OUT_CH, HID), full(1, HID), full(1, HID), full(1, HID),
            full(HID, OUT_CH), full(1, OUT_CH),
        ],
        out_specs=pl.BlockSpec((QB, OUT_CH), lambda i: (i, 0)),
        out_shape=jax.ShapeDtypeStruct((NHAT, OUT_CH), _F32),
        compiler_params=pltpu.CompilerParams(
            dimension_semantics=("parallel",)),
    )(g3, s_all, we2, be2, ge, bte, wo1, bo1, go, bto, wo2, bo2)


def kernel(vertices, features, We1, be1, ge, bte, We2, be2,
           Wo1, bo1, go, bto, Wo2, bo2):
    v = vertices[0]
    f = features[0]
    pad = NHAT - N
    v_pad = jnp.concatenate(
        [v, jnp.full((pad, 3), 100.0, _F32)], axis=0)
    f_pad = jnp.concatenate([f, jnp.zeros((pad, IN_CH), _F32)], axis=0)
    vt = v_pad.T

    wnb = We1[:IN_CH]
    wself = We1[IN_CH:2 * IN_CH]
    wrel = We1[2 * IN_CH:]

    p_tab, s_all = _proj(f_pad, v_pad, wnb, wself, wrel,
                         be1.reshape(1, HID))
    idx = _knn(v_pad, vt)
    idxt = idx.T.reshape(-1)
    g = _gather_rows(p_tab, idxt)
    g3 = g.reshape(K, NHAT, HID)
    out = _tail(g3, s_all, We2, be2.reshape(1, OUT_CH),
                ge.reshape(1, HID), bte.reshape(1, HID),
                Wo1, bo1.reshape(1, HID), go.reshape(1, HID),
                bto.reshape(1, HID), Wo2, bo2.reshape(1, OUT_CH))
    return out[:N][None]


# speedup vs baseline: 5.0318x; 1.0652x over previous
import functools

import jax
import jax.numpy as jnp
from jax import lax
from jax.experimental import pallas as pl
from jax.experimental.pallas import tpu as pltpu
from jax.experimental.pallas import tpu_sc as plsc

N = 10000
K = 16
IN_CH = 256
OUT_CH = 16
HID = 32
NHAT = 10240
QB = 256
NBLK = NHAT // QB
NW = 32
E = K * NHAT
EPW = E // NW
CH = 128
NCH = EPW // CH

_F32 = jnp.float32
_BF16 = jnp.bfloat16


def _proj_body(f_ref, v_ref, wnb_ref, wself_ref, wrel_ref, b1_ref,
               p_ref, s_ref):
    fb = f_ref[...].astype(_BF16)
    vb = v_ref[...].astype(_BF16)
    wnb = wnb_ref[...].astype(_BF16)
    wself = wself_ref[...].astype(_BF16)
    wrel = wrel_ref[...].astype(_BF16)
    vc = jnp.dot(vb, wrel, preferred_element_type=_F32)
    p_ref[...] = jnp.dot(fb, wnb, preferred_element_type=_F32) + vc
    s_ref[...] = (jnp.dot(fb, wself, preferred_element_type=_F32) - vc
                  + b1_ref[...])


def _proj(f_pad, v_pad, wnb, wself, wrel, b1):
    rb = 1024
    grid = NHAT // rb
    return pl.pallas_call(
        _proj_body,
        grid=(grid,),
        in_specs=[
            pl.BlockSpec((rb, IN_CH), lambda i: (i, 0)),
            pl.BlockSpec((rb, 3), lambda i: (i, 0)),
            pl.BlockSpec((IN_CH, HID), lambda i: (0, 0)),
            pl.BlockSpec((IN_CH, HID), lambda i: (0, 0)),
            pl.BlockSpec((3, HID), lambda i: (0, 0)),
            pl.BlockSpec((1, HID), lambda i: (0, 0)),
        ],
        out_specs=[
            pl.BlockSpec((rb, HID), lambda i: (i, 0)),
            pl.BlockSpec((rb, HID), lambda i: (i, 0)),
        ],
        out_shape=[
            jax.ShapeDtypeStruct((NHAT, HID), _F32),
            jax.ShapeDtypeStruct((NHAT, HID), _F32),
        ],
        compiler_params=pltpu.CompilerParams(
            dimension_semantics=("parallel",)),
    )(f_pad, v_pad, wnb, wself, wrel, b1)


def _knn_body(q_ref, vt_ref, idx_ref):
    qx = q_ref[:, 0:1]
    qy = q_ref[:, 1:2]
    qz = q_ref[:, 2:3]
    vx = vt_ref[0:1, :]
    vy = vt_ref[1:2, :]
    vz = vt_ref[2:3, :]
    qsq = qx * qx + qy * qy + qz * qz
    vsq = vx * vx + vy * vy + vz * vz

    dot = jnp.dot(q_ref[...].astype(_BF16), vt_ref[...].astype(_BF16),
                  preferred_element_type=_F32)
    d = (qsq + vsq) - 2.0 * dot

    iota = lax.broadcasted_iota(jnp.int32, (QB, NHAT), 1)
    cols = []
    for t in range(K):
        j = jnp.argmin(d, axis=1).astype(jnp.int32)
        cols.append(j)
        if t < K - 1:
            d = jnp.where(iota == j[:, None], jnp.inf, d)
    idx_ref[...] = jnp.stack(cols, axis=1)


def _knn(v_pad, vt):
    return pl.pallas_call(
        _knn_body,
        grid=(NBLK,),
        in_specs=[
            pl.BlockSpec((QB, 3), lambda i: (i, 0)),
            pl.BlockSpec((3, NHAT), lambda i: (0, 0)),
        ],
        out_specs=pl.BlockSpec((QB, K), lambda i: (i, 0)),
        out_shape=jax.ShapeDtypeStruct((NHAT, K), jnp.int32),
        compiler_params=pltpu.CompilerParams(
            dimension_semantics=("parallel",)),
    )(v_pad, vt)


def _gather_rows(table, idxt):
    mesh = plsc.VectorSubcoreMesh(core_axis_name="c", subcore_axis_name="s")

    @functools.partial(
        pl.kernel,
        mesh=mesh,
        out_type=jax.ShapeDtypeStruct((E, HID), _F32),
        scratch_types=[
            pltpu.VMEM((EPW,), jnp.int32),
            pltpu.VMEM((CH, HID), _F32),
            pltpu.SemaphoreType.DMA,
        ],
        compiler_params=pltpu.CompilerParams(use_tc_tiling_on_sc=False),
    )
    def gk(idx_hbm, tab_hbm, out_hbm, idx_v, buf, sem):
        wid = lax.axis_index("s") * 2 + lax.axis_index("c")
        base = wid * EPW
        pltpu.sync_copy(idx_hbm.at[pl.ds(base, EPW)], idx_v)

        def body(j, carry):
            pltpu.async_copy(
                tab_hbm.at[idx_v.at[pl.ds(j * CH, CH)]], buf, sem).wait()
            pltpu.sync_copy(buf, out_hbm.at[pl.ds(base + j * CH, CH)])
            return carry

        lax.fori_loop(0, NCH, body, 0)

    return gk(idxt, table)


def _tail_body(g_ref, s_ref, we2_ref, be2_ref, ge_ref, bte_ref,
               wo1_ref, bo1_ref, go_ref, bto_ref, wo2_ref, bo2_ref,
               out_ref):
    s = s_ref[...]
    we2 = we2_ref[...].astype(_BF16)
    ge = ge_ref[...]
    bte = bte_ref[...]
    acc = jnp.zeros((QB, OUT_CH), _F32)
    for k in range(K):
        h = g_ref[k] + s
        mu = jnp.mean(h, axis=-1, keepdims=True)
        var = jnp.mean((h - mu) ** 2, axis=-1, keepdims=True)
        h = (h - mu) * lax.rsqrt(var + 1e-5) * ge + bte
        h = jax.nn.gelu(h)
        acc = acc + jnp.dot(h.astype(_BF16), we2, preferred_element_type=_F32)
    agg = acc * (1.0 / K) + be2_ref[...]

    t = jnp.dot(agg.astype(_BF16), wo1_ref[...].astype(_BF16),
                preferred_element_type=_F32) + bo1_ref[...]
    mu = jnp.mean(t, axis=-1, keepdims=True)
    var = jnp.mean((t - mu) ** 2, axis=-1, keepdims=True)
    t = (t - mu) * lax.rsqrt(var + 1e-5) * go_ref[...] + bto_ref[...]
    t = jax.nn.gelu(t)
    out_ref[...] = jnp.dot(t.astype(_BF16), wo2_ref[...].astype(_BF16),
                           preferred_element_type=_F32) + bo2_ref[...]


def _tail(g3, s_all, we2, be2, ge, bte, wo1, bo1, go, bto, wo2, bo2):
    full = lambda a, b: pl.BlockSpec((a, b), lambda i: (0, 0))
    return pl.pallas_call(
        _tail_body,
        grid=(NBLK,),
        in_specs=[
            pl.BlockSpec((K, QB, HID), lambda i: (0, i, 0)),
            pl.BlockSpec((QB, HID), lambda i: (i, 0)),
            full(HID, OUT_CH), full(1, OUT_CH), full(1, HID), full(1, HID),
            full(OUT_CH, HID), full(1, HID), full(1, HID), full(1, HID),
            full(HID, OUT_CH), full(1, OUT_CH),
        ],
        out_specs=pl.BlockSpec((QB, OUT_CH), lambda i: (i, 0)),
        out_shape=jax.ShapeDtypeStruct((NHAT, OUT_CH), _F32),
        compiler_params=pltpu.CompilerParams(
            dimension_semantics=("parallel",)),
    )(g3, s_all, we2, be2, ge, bte, wo1, bo1, go, bto, wo2, bo2)


def kernel(vertices, features, We1, be1, ge, bte, We2, be2,
           Wo1, bo1, go, bto, Wo2, bo2):
    v = vertices[0]
    f = features[0]
    pad = NHAT - N
    v_pad = jnp.concatenate(
        [v, jnp.full((pad, 3), 100.0, _F32)], axis=0)
    f_pad = jnp.concatenate([f, jnp.zeros((pad, IN_CH), _F32)], axis=0)
    vt = v_pad.T

    wnb = We1[:IN_CH]
    wself = We1[IN_CH:2 * IN_CH]
    wrel = We1[2 * IN_CH:]

    p_tab, s_all = _proj(f_pad, v_pad, wnb, wself, wrel,
                         be1.reshape(1, HID))
    idx = _knn(v_pad, vt)
    idxt = idx.T.reshape(-1)
    g = _gather_rows(p_tab, idxt)
    g3 = g.reshape(K, NHAT, HID)
    out = _tail(g3, s_all, We2, be2.reshape(1, OUT_CH),
                ge.reshape(1, HID), bte.reshape(1, HID),
                Wo1, bo1.reshape(1, HID), go.reshape(1, HID),
                bto.reshape(1, HID), Wo2, bo2.reshape(1, OUT_CH))
    return out[:N][None]
